# Initial kernel scaffold; baseline (speedup 1.0000x reference)
import functools

import jax
import jax.numpy as jnp
from jax import lax
from jax.experimental import pallas as pl
from jax.experimental.pallas import tpu as pltpu
from jax.experimental.pallas import tpu_sc as plsc

N = 200000
D = 64
K = 16
SECOND_MATCH_FACTOR = 3.0
DIST_LIMIT = 20.0 * D

NW = 32
ROWS_W = N // NW
CHUNK = 250
NCHUNKS = ROWS_W // CHUNK
IDX_PAD = ROWS_W + 6

_sc_mesh = plsc.VectorSubcoreMesh(core_axis_name="c", subcore_axis_name="s")


@functools.partial(
    pl.kernel,
    mesh=_sc_mesh,
    out_type=[
        jax.ShapeDtypeStruct((NW, K, D), jnp.float32),
        jax.ShapeDtypeStruct((NW, K, D), jnp.float32),
        jax.ShapeDtypeStruct((NW, K), jnp.float32),
    ],
    scratch_types=[
        pltpu.VMEM((IDX_PAD,), jnp.int32),
        pltpu.VMEM((2, CHUNK, D), jnp.float32),
        pltpu.VMEM((K, D), jnp.float32),
        pltpu.VMEM((K, D), jnp.float32),
        pltpu.VMEM((K,), jnp.float32),
        pltpu.SemaphoreType.DMA,
        pltpu.SemaphoreType.DMA,
        pltpu.SemaphoreType.DMA,
    ],
)
def _segment_sums_sc(spikes_hbm, sidx_hbm, sums_hbm, sumsq_hbm, counts_hbm,
                     idx_v, buf_v, sums_v, sumsq_v, cnt_v, sem0, sem1, sem2):
    cid = lax.axis_index("c")
    sid = lax.axis_index("s")
    w = sid * 2 + cid
    base = w * ROWS_W

    cp_idx = pltpu.async_copy(sidx_hbm.at[w], idx_v, sem2)

    zeros16 = jnp.zeros((16,), jnp.float32)
    for i in range(K):
        for j in range(D // 16):
            sums_v[i, pl.ds(j * 16, 16)] = zeros16
            sumsq_v[i, pl.ds(j * 16, 16)] = zeros16
    cp_idx.wait()

    sems = (sem0, sem1)
    copies = [
        pltpu.async_copy(spikes_hbm.at[pl.ds(base, CHUNK), :], buf_v.at[0],
                         sems[0]),
        None,
    ]
    cnt = jnp.zeros((16,), jnp.float32)
    lanes = lax.iota(jnp.int32, 16)
    for k in range(NCHUNKS):
        b = k % 2
        copies[b].wait()
        if k + 1 < NCHUNKS:
            nb = (k + 1) % 2
            copies[nb] = pltpu.async_copy(
                spikes_hbm.at[pl.ds(base + (k + 1) * CHUNK, CHUNK), :],
                buf_v.at[nb], sems[nb])

        def row_body(r, cnt, b=b, k=k):
            cidx = idx_v[k * CHUNK + r]
            for j in range(D // 16):
                v = buf_v[b, r, pl.ds(j * 16, 16)]
                plsc.addupdate(sums_v.at[cidx, pl.ds(j * 16, 16)], v)
                plsc.addupdate(sumsq_v.at[cidx, pl.ds(j * 16, 16)], v * v)
            return cnt + jnp.where(lanes == cidx, 1.0, 0.0)

        cnt = lax.fori_loop(0, CHUNK, row_body, cnt)

    cnt_v[...] = cnt
    pltpu.sync_copy(sums_v, sums_hbm.at[w])
    pltpu.sync_copy(sumsq_v, sumsq_hbm.at[w])
    pltpu.sync_copy(cnt_v, counts_hbm.at[w])


B = 4000
GRID = N // B


def _match_body(spikes_ref, sidx_ref, midx_ref, sums_ref, sumsq_ref,
                counts_ref, osort_ref, omatch_ref, odist_ref):
    counts = jnp.sum(counts_ref[...], axis=0)
    sums = jnp.sum(sums_ref[...], axis=0)
    sumsq = jnp.sum(sumsq_ref[...], axis=0)
    cnt = jnp.maximum(counts, 1.0)
    means = sums / cnt[:, None]
    var = sumsq / cnt[:, None] - means * means
    var_sum = jnp.maximum(jnp.sum(var, axis=1), 0.0)
    stds = jnp.sqrt(var_sum)
    m2 = jnp.sum(means * means, axis=1)

    x = spikes_ref[...]
    x2 = jnp.sum(x * x, axis=1, keepdims=True)
    dots = lax.dot_general(x, means, (((1,), (1,)), ((), ())),
                           preferred_element_type=jnp.float32)
    d2 = x2 + m2[None, :] - 2.0 * dots
    dist = jnp.sqrt(jnp.maximum(d2, 0.0))
    thr = SECOND_MATCH_FACTOR * stds
    colid = lax.broadcasted_iota(jnp.int32, (B, K), 1)
    masked = jnp.where(dist > thr[None, :], jnp.inf, dist)
    masked = jnp.where(colid == 0, jnp.inf, masked)
    min_vals = jnp.min(masked, axis=1, keepdims=True)
    am = jnp.min(jnp.where(masked == min_vals, colid, K), axis=1,
                 keepdims=True)
    minimizer = jnp.where(min_vals >= DIST_LIMIT, 0, am).astype(jnp.int32)

    sidx = sidx_ref[...]
    mask = sidx == 0
    osort_ref[...] = jnp.where(mask, minimizer, sidx)
    omatch_ref[...] = jnp.where(mask, 2, midx_ref[...])
    odist_ref[...] = jnp.where(mask, min_vals, 0.0)


def _match_tc(spikes, sidx2, midx2, sums, sumsq, counts):
    out_shape = [
        jax.ShapeDtypeStruct((N, 1), jnp.int32),
        jax.ShapeDtypeStruct((N, 1), jnp.int32),
        jax.ShapeDtypeStruct((N, 1), jnp.float32),
    ]
    return pl.pallas_call(
        _match_body,
        grid=(GRID,),
        in_specs=[
            pl.BlockSpec((B, D), lambda i: (i, 0)),
            pl.BlockSpec((B, 1), lambda i: (i, 0)),
            pl.BlockSpec((B, 1), lambda i: (i, 0)),
            pl.BlockSpec((NW, K, D), lambda i: (0, 0, 0)),
            pl.BlockSpec((NW, K, D), lambda i: (0, 0, 0)),
            pl.BlockSpec((NW, K), lambda i: (0, 0)),
        ],
        out_specs=[
            pl.BlockSpec((B, 1), lambda i: (i, 0)),
            pl.BlockSpec((B, 1), lambda i: (i, 0)),
            pl.BlockSpec((B, 1), lambda i: (i, 0)),
        ],
        out_shape=out_shape,
        compiler_params=pltpu.CompilerParams(
            dimension_semantics=("arbitrary",)),
    )(spikes, sidx2, midx2, sums, sumsq, counts)


def kernel(spikes, sort_idx, match_idx):
    sidx_rows = jnp.pad(sort_idx.reshape(NW, ROWS_W), ((0, 0), (0, 6)))
    sums, sumsq, counts = _segment_sums_sc(spikes, sidx_rows)
    sidx2 = sort_idx.reshape(N, 1)
    midx2 = match_idx.reshape(N, 1)
    ns, nm, dv = _match_tc(spikes, sidx2, midx2, sums, sumsq, counts)
    return (ns.reshape(N), nm.reshape(N), dv.reshape(N))


# speedup vs baseline: 1.9998x; 1.9998x over previous
import functools

import jax
import jax.numpy as jnp
from jax import lax
from jax.experimental import pallas as pl
from jax.experimental.pallas import tpu as pltpu
from jax.experimental.pallas import tpu_sc as plsc

N = 200000
D = 64
K = 16
SECOND_MATCH_FACTOR = 3.0
DIST_LIMIT = 20.0 * D

NW = 32
ROWS_W = N // NW
CHUNK = 240
NCHUNKS = ROWS_W // CHUNK
NPAIRS = NCHUNKS // 2
TAIL = ROWS_W - NCHUNKS * CHUNK
IDX_PAD = ROWS_W + 6

_sc_mesh = plsc.VectorSubcoreMesh(core_axis_name="c", subcore_axis_name="s")


@functools.partial(
    pl.kernel,
    mesh=_sc_mesh,
    out_type=[
        jax.ShapeDtypeStruct((NW * K * D,), jnp.float32),
        jax.ShapeDtypeStruct((NW * K * 16,), jnp.float32),
        jax.ShapeDtypeStruct((NW * K,), jnp.float32),
    ],
    scratch_types=[
        pltpu.VMEM((IDX_PAD,), jnp.int32),
        pltpu.VMEM((2, CHUNK * D), jnp.float32),
        pltpu.VMEM((K * D,), jnp.float32),
        pltpu.VMEM((K * 16,), jnp.float32),
        pltpu.VMEM((16,), jnp.float32),
        pltpu.SemaphoreType.DMA,
        pltpu.SemaphoreType.DMA,
        pltpu.SemaphoreType.DMA,
    ],
)
def _segment_sums_sc(spikes_hbm, sidx_hbm, sums_hbm, sqn_hbm, counts_hbm,
                     idx_v, buf_v, sums_v, sqn_v, cnt_v, sem0, sem1, sem2):
    cid = lax.axis_index("c")
    sid = lax.axis_index("s")
    w = sid * 2 + cid
    base = w * ROWS_W

    cp_idx = pltpu.async_copy(sidx_hbm.at[pl.ds(w * IDX_PAD, IDX_PAD)], idx_v,
                              sem2)

    zeros16 = jnp.zeros((16,), jnp.float32)
    for i in range(K * D // 16):
        sums_v[pl.ds(i * 16, 16)] = zeros16
    for i in range(K):
        sqn_v[pl.ds(i * 16, 16)] = zeros16
    cp_idx.wait()

    sems = (sem0, sem1)
    pltpu.async_copy(spikes_hbm.at[pl.ds(base * D, CHUNK * D)], buf_v.at[0],
                     sems[0])
    pltpu.async_copy(spikes_hbm.at[pl.ds((base + CHUNK) * D, CHUNK * D)],
                     buf_v.at[1], sems[1])
    lanes = lax.iota(jnp.int32, 16)

    def _consume_rows(cnt, b, row0, idx0, nrows):
        civ = idx_v[pl.ds(idx0, 16)]
        for l in range(nrows):
            cidx = civ[l]
            off = (row0 + l) * D
            v0 = buf_v[b, pl.ds(off, 16)]
            v1 = buf_v[b, pl.ds(off + 16, 16)]
            v2 = buf_v[b, pl.ds(off + 32, 16)]
            v3 = buf_v[b, pl.ds(off + 48, 16)]
            sq = v0 * v0 + v1 * v1 + v2 * v2 + v3 * v3
            coff = cidx * D
            plsc.addupdate(sums_v.at[pl.ds(coff, 16)], v0)
            plsc.addupdate(sums_v.at[pl.ds(coff + 16, 16)], v1)
            plsc.addupdate(sums_v.at[pl.ds(coff + 32, 16)], v2)
            plsc.addupdate(sums_v.at[pl.ds(coff + 48, 16)], v3)
            plsc.addupdate(sqn_v.at[pl.ds(cidx * 16, 16)], sq)
            cnt = cnt + jnp.where(lanes == cidx, 1.0, 0.0)
        return cnt

    def pair_body(p, cnt):
        for b in range(2):
            c_idx = 2 * p + b
            pltpu.make_async_copy(
                spikes_hbm.at[pl.ds((base + c_idx * CHUNK) * D, CHUNK * D)],
                buf_v.at[b], sems[b]).wait()

            def grp_body(g, cnt, b=b, c_idx=c_idx):
                return _consume_rows(cnt, b, g * 16, c_idx * CHUNK + g * 16,
                                     16)

            cnt = lax.fori_loop(0, CHUNK // 16, grp_body, cnt)

            @pl.when(c_idx + 2 < NCHUNKS)
            def _(b=b, c_idx=c_idx):
                pltpu.async_copy(
                    spikes_hbm.at[
                        pl.ds((base + (c_idx + 2) * CHUNK) * D, CHUNK * D)],
                    buf_v.at[b], sems[b])
        return cnt

    cnt = lax.fori_loop(0, NPAIRS, pair_body, jnp.zeros((16,), jnp.float32))

    pltpu.async_copy(
        spikes_hbm.at[pl.ds((base + NCHUNKS * CHUNK) * D, TAIL * D)],
        buf_v.at[0, pl.ds(0, TAIL * D)], sems[0]).wait()
    cnt = _consume_rows(cnt, 0, 0, NCHUNKS * CHUNK, TAIL)

    cnt_v[...] = cnt
    pltpu.sync_copy(sums_v, sums_hbm.at[pl.ds(w * K * D, K * D)])
    pltpu.sync_copy(sqn_v, sqn_hbm.at[pl.ds(w * K * 16, K * 16)])
    pltpu.sync_copy(cnt_v, counts_hbm.at[pl.ds(w * K, K)])


B = 4000
GRID = N // B


def _match_body(spikes_ref, sidx_ref, midx_ref, sums_ref, sqn_ref,
                counts_ref, osort_ref, omatch_ref, odist_ref):
    counts = jnp.sum(counts_ref[...], axis=0)
    sums = jnp.sum(sums_ref[...], axis=0)
    sqn = jnp.sum(sqn_ref[...], axis=(0, 2))
    cnt = jnp.maximum(counts, 1.0)
    means = sums / cnt[:, None]
    m2 = jnp.sum(means * means, axis=1)
    var_sum = jnp.maximum(sqn / cnt - m2, 0.0)
    stds = jnp.sqrt(var_sum)

    x = spikes_ref[...]
    x2 = jnp.sum(x * x, axis=1, keepdims=True)
    dots = lax.dot_general(x, means, (((1,), (1,)), ((), ())),
                           preferred_element_type=jnp.float32)
    d2 = x2 + m2[None, :] - 2.0 * dots
    dist = jnp.sqrt(jnp.maximum(d2, 0.0))
    thr = SECOND_MATCH_FACTOR * stds
    colid = lax.broadcasted_iota(jnp.int32, (B, K), 1)
    masked = jnp.where(dist > thr[None, :], jnp.inf, dist)
    masked = jnp.where(colid == 0, jnp.inf, masked)
    min_vals = jnp.min(masked, axis=1, keepdims=True)
    am = jnp.min(jnp.where(masked == min_vals, colid, K), axis=1,
                 keepdims=True)
    minimizer = jnp.where(min_vals >= DIST_LIMIT, 0, am).astype(jnp.int32)

    sidx = sidx_ref[...]
    mask = sidx == 0
    osort_ref[...] = jnp.where(mask, minimizer, sidx)
    omatch_ref[...] = jnp.where(mask, 2, midx_ref[...])
    odist_ref[...] = jnp.where(mask, min_vals, 0.0)


def _match_tc(spikes, sidx2, midx2, sums, sqn, counts):
    out_shape = [
        jax.ShapeDtypeStruct((N, 1), jnp.int32),
        jax.ShapeDtypeStruct((N, 1), jnp.int32),
        jax.ShapeDtypeStruct((N, 1), jnp.float32),
    ]
    return pl.pallas_call(
        _match_body,
        grid=(GRID,),
        in_specs=[
            pl.BlockSpec((B, D), lambda i: (i, 0)),
            pl.BlockSpec((B, 1), lambda i: (i, 0)),
            pl.BlockSpec((B, 1), lambda i: (i, 0)),
            pl.BlockSpec((NW, K, D), lambda i: (0, 0, 0)),
            pl.BlockSpec((NW, K, 16), lambda i: (0, 0, 0)),
            pl.BlockSpec((NW, K), lambda i: (0, 0)),
        ],
        out_specs=[
            pl.BlockSpec((B, 1), lambda i: (i, 0)),
            pl.BlockSpec((B, 1), lambda i: (i, 0)),
            pl.BlockSpec((B, 1), lambda i: (i, 0)),
        ],
        out_shape=out_shape,
        compiler_params=pltpu.CompilerParams(
            dimension_semantics=("arbitrary",)),
    )(spikes, sidx2, midx2, sums, sqn, counts)


def kernel(spikes, sort_idx, match_idx):
    sidx_flat = jnp.pad(sort_idx.reshape(NW, ROWS_W),
                        ((0, 0), (0, IDX_PAD - ROWS_W))).reshape(-1)
    sums_f, sqn_f, counts_f = _segment_sums_sc(spikes.reshape(-1), sidx_flat)
    sums = sums_f.reshape(NW, K, D)
    sqn = sqn_f.reshape(NW, K, 16)
    counts = counts_f.reshape(NW, K)
    sidx2 = sort_idx.reshape(N, 1)
    midx2 = match_idx.reshape(N, 1)
    ns, nm, dv = _match_tc(spikes, sidx2, midx2, sums, sqn, counts)
    return (ns.reshape(N), nm.reshape(N), dv.reshape(N))


# speedup vs baseline: 4.1635x; 2.0819x over previous
import functools

import jax
import jax.numpy as jnp
from jax import lax
from jax.experimental import pallas as pl
from jax.experimental.pallas import tpu as pltpu
from jax.experimental.pallas import tpu_sc as plsc

N = 200000
D = 64
K = 16
SECOND_MATCH_FACTOR = 3.0
DIST_LIMIT = 20.0 * D

NW = 32
ROWS_W = N // NW
CHUNK = 240
NCHUNKS = ROWS_W // CHUNK
NPAIRS = NCHUNKS // 2
TAIL = ROWS_W - NCHUNKS * CHUNK
IDX_PAD = ROWS_W + 6

_sc_mesh = plsc.VectorSubcoreMesh(core_axis_name="c", subcore_axis_name="s")


@functools.partial(
    pl.kernel,
    mesh=_sc_mesh,
    out_type=[
        jax.ShapeDtypeStruct((NW * K * D,), jnp.float32),
        jax.ShapeDtypeStruct((NW * K * 16,), jnp.float32),
        jax.ShapeDtypeStruct((NW * K,), jnp.float32),
    ],
    scratch_types=[
        pltpu.VMEM((IDX_PAD,), jnp.int32),
        pltpu.VMEM((2, CHUNK * D), jnp.float32),
        pltpu.VMEM((K * D,), jnp.float32),
        pltpu.VMEM((K * 16,), jnp.float32),
        pltpu.VMEM((16,), jnp.float32),
        pltpu.SemaphoreType.DMA,
        pltpu.SemaphoreType.DMA,
        pltpu.SemaphoreType.DMA,
    ],
)
def _segment_sums_sc(spikes_hbm, sidx_hbm, sums_hbm, sqn_hbm, counts_hbm,
                     idx_v, buf_v, sums_v, sqn_v, cnt_v, sem0, sem1, sem2):
    cid = lax.axis_index("c")
    sid = lax.axis_index("s")
    w = sid * 2 + cid
    base = w * ROWS_W

    cp_idx = pltpu.async_copy(sidx_hbm.at[pl.ds(w * IDX_PAD, IDX_PAD)], idx_v,
                              sem2)

    zeros16 = jnp.zeros((16,), jnp.float32)
    for i in range(K * D // 16):
        sums_v[pl.ds(i * 16, 16)] = zeros16
    for i in range(K):
        sqn_v[pl.ds(i * 16, 16)] = zeros16
    cp_idx.wait()

    sems = (sem0, sem1)
    pltpu.async_copy(spikes_hbm.at[pl.ds(base * D, CHUNK * D)], buf_v.at[0],
                     sems[0])
    pltpu.async_copy(spikes_hbm.at[pl.ds((base + CHUNK) * D, CHUNK * D)],
                     buf_v.at[1], sems[1])
    lanes = lax.iota(jnp.int32, 16)

    def _consume_rows(cnt, b, row0, idx0, nrows):
        civ = idx_v[pl.ds(idx0, 16)]
        for l in range(nrows):
            cidx = civ[l]
            off = (row0 + l) * D
            v0 = buf_v[b, pl.ds(off, 16)]
            v1 = buf_v[b, pl.ds(off + 16, 16)]
            v2 = buf_v[b, pl.ds(off + 32, 16)]
            v3 = buf_v[b, pl.ds(off + 48, 16)]
            sq = v0 * v0 + v1 * v1 + v2 * v2 + v3 * v3
            coff = cidx * D
            plsc.addupdate(sums_v.at[pl.ds(coff, 16)], v0)
            plsc.addupdate(sums_v.at[pl.ds(coff + 16, 16)], v1)
            plsc.addupdate(sums_v.at[pl.ds(coff + 32, 16)], v2)
            plsc.addupdate(sums_v.at[pl.ds(coff + 48, 16)], v3)
            plsc.addupdate(sqn_v.at[pl.ds(cidx * 16, 16)], sq)
            cnt = cnt + jnp.where(lanes == cidx, 1.0, 0.0)
        return cnt

    def pair_body(p, cnt):
        for b in range(2):
            c_idx = 2 * p + b
            pltpu.make_async_copy(
                spikes_hbm.at[pl.ds((base + c_idx * CHUNK) * D, CHUNK * D)],
                buf_v.at[b], sems[b]).wait()

            def grp_body(g, cnt, b=b, c_idx=c_idx):
                return _consume_rows(cnt, b, g * 16, c_idx * CHUNK + g * 16,
                                     16)

            cnt = lax.fori_loop(0, CHUNK // 16, grp_body, cnt)

            @pl.when(c_idx + 2 < NCHUNKS)
            def _(b=b, c_idx=c_idx):
                pltpu.async_copy(
                    spikes_hbm.at[
                        pl.ds((base + (c_idx + 2) * CHUNK) * D, CHUNK * D)],
                    buf_v.at[b], sems[b])
        return cnt

    cnt = lax.fori_loop(0, NPAIRS, pair_body, jnp.zeros((16,), jnp.float32))

    pltpu.async_copy(
        spikes_hbm.at[pl.ds((base + NCHUNKS * CHUNK) * D, TAIL * D)],
        buf_v.at[0, pl.ds(0, TAIL * D)], sems[0]).wait()
    cnt = _consume_rows(cnt, 0, 0, NCHUNKS * CHUNK, TAIL)

    cnt_v[...] = cnt
    pltpu.sync_copy(sums_v, sums_hbm.at[pl.ds(w * K * D, K * D)])
    pltpu.sync_copy(sqn_v, sqn_hbm.at[pl.ds(w * K * 16, K * 16)])
    pltpu.sync_copy(cnt_v, counts_hbm.at[pl.ds(w * K, K)])


B = 8000
GRID = N // B
LC = 500
SR = B // LC


def _match_body(spikes_ref, sidx_ref, midx_ref, sums_ref, sqn_ref,
                counts_ref, osort_ref, omatch_ref, odist_ref):
    counts = jnp.sum(counts_ref[...], axis=0)
    sums = jnp.sum(sums_ref[...], axis=0)
    sqn = jnp.sum(sqn_ref[...], axis=(0, 2))
    cnt = jnp.maximum(counts, 1.0)
    means = sums / cnt[:, None]
    m2 = jnp.sum(means * means, axis=1)
    var_sum = jnp.maximum(sqn / cnt - m2, 0.0)
    thr = SECOND_MATCH_FACTOR * jnp.sqrt(var_sum)
    thr2 = thr * thr

    x = spikes_ref[...]
    dots2_t = lax.dot_general(means * -2.0, x, (((1,), (1,)), ((), ())),
                              preferred_element_type=jnp.float32)
    xx = x * x
    x2_t = lax.dot_general(jnp.ones((1, D), jnp.float32), xx,
                           (((1,), (1,)), ((), ())),
                           preferred_element_type=jnp.float32)
    d2 = x2_t + m2[:, None] + dots2_t
    rowid = lax.broadcasted_iota(jnp.int32, (K, B), 0)
    bad = (d2 > thr2[:, None]) | (rowid == 0)
    masked = jnp.where(bad, jnp.inf, d2)
    min_d2 = jnp.min(masked, axis=0, keepdims=True)
    am = jnp.min(jnp.where(masked == min_d2, rowid, K), axis=0,
                 keepdims=True)
    min_vals = jnp.sqrt(jnp.maximum(min_d2, 0.0))
    minimizer = jnp.where(min_vals >= DIST_LIMIT, 0, am).astype(jnp.int32)

    def _to_rows(v):
        return jnp.concatenate(
            [lax.slice(v, (0, s * LC), (1, (s + 1) * LC)) for s in range(SR)],
            axis=0)

    mz = _to_rows(minimizer)
    mv = _to_rows(min_vals)
    sidx = sidx_ref[...]
    mask = sidx == 0
    osort_ref[...] = jnp.where(mask, mz, sidx)
    omatch_ref[...] = jnp.where(mask, 2, midx_ref[...])
    odist_ref[...] = jnp.where(mask, mv, 0.0)


def _match_tc(spikes, sidx2, midx2, sums, sqn, counts):
    out_shape = [
        jax.ShapeDtypeStruct((N // LC, LC), jnp.int32),
        jax.ShapeDtypeStruct((N // LC, LC), jnp.int32),
        jax.ShapeDtypeStruct((N // LC, LC), jnp.float32),
    ]
    return pl.pallas_call(
        _match_body,
        grid=(GRID,),
        in_specs=[
            pl.BlockSpec((B, D), lambda i: (i, 0)),
            pl.BlockSpec((SR, LC), lambda i: (i, 0)),
            pl.BlockSpec((SR, LC), lambda i: (i, 0)),
            pl.BlockSpec((NW, K, D), lambda i: (0, 0, 0)),
            pl.BlockSpec((NW, K, 16), lambda i: (0, 0, 0)),
            pl.BlockSpec((NW, K), lambda i: (0, 0)),
        ],
        out_specs=[
            pl.BlockSpec((SR, LC), lambda i: (i, 0)),
            pl.BlockSpec((SR, LC), lambda i: (i, 0)),
            pl.BlockSpec((SR, LC), lambda i: (i, 0)),
        ],
        out_shape=out_shape,
        compiler_params=pltpu.CompilerParams(
            dimension_semantics=("arbitrary",)),
    )(spikes, sidx2, midx2, sums, sqn, counts)


def kernel(spikes, sort_idx, match_idx):
    sidx_flat = jnp.pad(sort_idx.reshape(NW, ROWS_W),
                        ((0, 0), (0, IDX_PAD - ROWS_W))).reshape(-1)
    sums_f, sqn_f, counts_f = _segment_sums_sc(spikes.reshape(-1), sidx_flat)
    sums = sums_f.reshape(NW, K, D)
    sqn = sqn_f.reshape(NW, K, 16)
    counts = counts_f.reshape(NW, K)
    sidx2 = sort_idx.reshape(N // LC, LC)
    midx2 = match_idx.reshape(N // LC, LC)
    ns, nm, dv = _match_tc(spikes, sidx2, midx2, sums, sqn, counts)
    return (ns.reshape(N), nm.reshape(N), dv.reshape(N))


# speedup vs baseline: 5.9978x; 1.4406x over previous
import functools

import jax
import jax.numpy as jnp
from jax import lax
from jax.experimental import pallas as pl
from jax.experimental.pallas import tpu as pltpu
from jax.experimental.pallas import tpu_sc as plsc

N = 200000
D = 64
K = 16
SECOND_MATCH_FACTOR = 3.0
DIST_LIMIT = 20.0 * D

NW = 32
ROWS_BASE = 6248
EXTRA = N - NW * ROWS_BASE
CHUNK = 240
NCHUNKS = ROWS_BASE // CHUNK
NPAIRS = NCHUNKS // 2
TAIL = ROWS_BASE - NCHUNKS * CHUNK
IDX_LOAD = 6256

_sc_mesh = plsc.VectorSubcoreMesh(core_axis_name="c", subcore_axis_name="s")


@functools.partial(
    pl.kernel,
    mesh=_sc_mesh,
    out_type=[
        jax.ShapeDtypeStruct((NW * K * D,), jnp.float32),
        jax.ShapeDtypeStruct((NW * K * 16,), jnp.float32),
        jax.ShapeDtypeStruct((NW * K,), jnp.float32),
    ],
    scratch_types=[
        pltpu.VMEM((ROWS_BASE + EXTRA + 8,), jnp.int32),
        pltpu.VMEM((2, CHUNK, D), jnp.float32),
        pltpu.VMEM((K * D,), jnp.float32),
        pltpu.VMEM((K * 16,), jnp.float32),
        pltpu.VMEM((16,), jnp.float32),
        pltpu.SemaphoreType.DMA,
        pltpu.SemaphoreType.DMA,
        pltpu.SemaphoreType.DMA,
    ],
)
def _segment_sums_sc(spikes_hbm, sidx_hbm, sums_hbm, sqn_hbm, counts_hbm,
                     idx_v, buf_v, sums_v, sqn_v, cnt_v, sem0, sem1, sem2):
    cid = lax.axis_index("c")
    sid = lax.axis_index("s")
    w = sid * 2 + cid
    base = w * ROWS_BASE
    is_last = w == NW - 1

    cp_idx = pltpu.async_copy(sidx_hbm.at[pl.ds(base, IDX_LOAD)],
                              idx_v.at[pl.ds(0, IDX_LOAD)], sem2)

    zeros16 = jnp.zeros((16,), jnp.float32)
    for i in range(K * D // 16):
        sums_v[pl.ds(i * 16, 16)] = zeros16
    for i in range(K):
        sqn_v[pl.ds(i * 16, 16)] = zeros16
    cp_idx.wait()

    @pl.when(is_last)
    def _():
        pltpu.async_copy(
            sidx_hbm.at[pl.ds(NW * ROWS_BASE - 8, EXTRA + 8)],
            idx_v.at[pl.ds(ROWS_BASE - 8, EXTRA + 8)], sem2).wait()

    sems = (sem0, sem1)
    pltpu.async_copy(spikes_hbm.at[pl.ds(base, CHUNK), :], buf_v.at[0],
                     sems[0])
    pltpu.async_copy(spikes_hbm.at[pl.ds(base + CHUNK, CHUNK), :],
                     buf_v.at[1], sems[1])
    lanes = lax.iota(jnp.int32, 16)

    def _consume_rows(cnt, b, row0, idx0, nrows):
        civ = idx_v[pl.ds(idx0, 16)]
        for l in range(nrows):
            cidx = civ[l]
            row = row0 + l
            v0 = buf_v[b, row, pl.ds(0, 16)]
            v1 = buf_v[b, row, pl.ds(16, 16)]
            v2 = buf_v[b, row, pl.ds(32, 16)]
            v3 = buf_v[b, row, pl.ds(48, 16)]
            sq = v0 * v0 + v1 * v1 + v2 * v2 + v3 * v3
            coff = cidx * D
            plsc.addupdate(sums_v.at[pl.ds(coff, 16)], v0)
            plsc.addupdate(sums_v.at[pl.ds(coff + 16, 16)], v1)
            plsc.addupdate(sums_v.at[pl.ds(coff + 32, 16)], v2)
            plsc.addupdate(sums_v.at[pl.ds(coff + 48, 16)], v3)
            plsc.addupdate(sqn_v.at[pl.ds(cidx * 16, 16)], sq)
            cnt = cnt + jnp.where(lanes == cidx, 1.0, 0.0)
        return cnt

    def pair_body(p, cnt):
        for b in range(2):
            c_idx = 2 * p + b
            pltpu.make_async_copy(
                spikes_hbm.at[pl.ds(base + c_idx * CHUNK, CHUNK), :],
                buf_v.at[b], sems[b]).wait()

            def grp_body(g, cnt, b=b, c_idx=c_idx):
                return _consume_rows(cnt, b, g * 16, c_idx * CHUNK + g * 16,
                                     16)

            cnt = lax.fori_loop(0, CHUNK // 16, grp_body, cnt)

            @pl.when(c_idx + 2 < NCHUNKS)
            def _(b=b, c_idx=c_idx):
                pltpu.async_copy(
                    spikes_hbm.at[
                        pl.ds(base + (c_idx + 2) * CHUNK, CHUNK), :],
                    buf_v.at[b], sems[b])
        return cnt

    cnt = lax.fori_loop(0, NPAIRS, pair_body, jnp.zeros((16,), jnp.float32))

    pltpu.async_copy(
        spikes_hbm.at[pl.ds(base + NCHUNKS * CHUNK, TAIL), :],
        buf_v.at[0, pl.ds(0, TAIL), :], sems[0]).wait()
    cnt = _consume_rows(cnt, 0, 0, NCHUNKS * CHUNK, TAIL)

    cnt_v[...] = cnt

    @pl.when(is_last)
    def _():
        pltpu.async_copy(
            spikes_hbm.at[pl.ds(NW * ROWS_BASE, EXTRA), :],
            buf_v.at[1, pl.ds(0, EXTRA), :], sems[1]).wait()

        def extra_grp(g, cnt):
            return _consume_rows(cnt, 1, g * 16, ROWS_BASE + g * 16, 16)

        cnt_v[...] = lax.fori_loop(0, EXTRA // 16, extra_grp, cnt_v[...])
    pltpu.sync_copy(sums_v, sums_hbm.at[pl.ds(w * K * D, K * D)])
    pltpu.sync_copy(sqn_v, sqn_hbm.at[pl.ds(w * K * 16, K * 16)])
    pltpu.sync_copy(cnt_v, counts_hbm.at[pl.ds(w * K, K)])


B = 8000
GRID = N // B
LC = 500
SR = B // LC


def _match_body(spikes_ref, sidx_ref, midx_ref, sums_ref, sqn_ref,
                counts_ref, osort_ref, omatch_ref, odist_ref):
    counts = jnp.sum(counts_ref[...], axis=0)
    sums = jnp.sum(sums_ref[...], axis=0)
    sqn = jnp.sum(sqn_ref[...], axis=(0, 2))
    cnt = jnp.maximum(counts, 1.0)
    means = sums / cnt[:, None]
    m2 = jnp.sum(means * means, axis=1)
    var_sum = jnp.maximum(sqn / cnt - m2, 0.0)
    thr = SECOND_MATCH_FACTOR * jnp.sqrt(var_sum)
    thr2 = thr * thr

    x = spikes_ref[...]
    dots2_t = lax.dot_general(means * -2.0, x, (((1,), (1,)), ((), ())),
                              preferred_element_type=jnp.float32)
    xx = x * x
    x2_t = lax.dot_general(jnp.ones((1, D), jnp.float32), xx,
                           (((1,), (1,)), ((), ())),
                           preferred_element_type=jnp.float32)
    d2 = x2_t + m2[:, None] + dots2_t
    rowid = lax.broadcasted_iota(jnp.int32, (K, B), 0)
    bad = (d2 > thr2[:, None]) | (rowid == 0)
    masked = jnp.where(bad, jnp.inf, d2)
    min_d2 = jnp.min(masked, axis=0, keepdims=True)
    am = jnp.min(jnp.where(masked == min_d2, rowid, K), axis=0,
                 keepdims=True)
    min_vals = jnp.sqrt(jnp.maximum(min_d2, 0.0))
    minimizer = jnp.where(min_vals >= DIST_LIMIT, 0, am).astype(jnp.int32)

    def _to_rows(v):
        return jnp.concatenate(
            [lax.slice(v, (0, s * LC), (1, (s + 1) * LC)) for s in range(SR)],
            axis=0)

    mz = _to_rows(minimizer)
    mv = _to_rows(min_vals)
    sidx = sidx_ref[...]
    mask = sidx == 0
    osort_ref[...] = jnp.where(mask, mz, sidx)
    omatch_ref[...] = jnp.where(mask, 2, midx_ref[...])
    odist_ref[...] = jnp.where(mask, mv, 0.0)


def _match_tc(spikes, sidx2, midx2, sums, sqn, counts):
    out_shape = [
        jax.ShapeDtypeStruct((N // LC, LC), jnp.int32),
        jax.ShapeDtypeStruct((N // LC, LC), jnp.int32),
        jax.ShapeDtypeStruct((N // LC, LC), jnp.float32),
    ]
    return pl.pallas_call(
        _match_body,
        grid=(GRID,),
        in_specs=[
            pl.BlockSpec((B, D), lambda i: (i, 0)),
            pl.BlockSpec((SR, LC), lambda i: (i, 0)),
            pl.BlockSpec((SR, LC), lambda i: (i, 0)),
            pl.BlockSpec((NW, K, D), lambda i: (0, 0, 0)),
            pl.BlockSpec((NW, K, 16), lambda i: (0, 0, 0)),
            pl.BlockSpec((NW, K), lambda i: (0, 0)),
        ],
        out_specs=[
            pl.BlockSpec((SR, LC), lambda i: (i, 0)),
            pl.BlockSpec((SR, LC), lambda i: (i, 0)),
            pl.BlockSpec((SR, LC), lambda i: (i, 0)),
        ],
        out_shape=out_shape,
        compiler_params=pltpu.CompilerParams(
            dimension_semantics=("arbitrary",)),
    )(spikes, sidx2, midx2, sums, sqn, counts)


def kernel(spikes, sort_idx, match_idx):
    sums_f, sqn_f, counts_f = _segment_sums_sc(spikes, sort_idx)
    sums = sums_f.reshape(NW, K, D)
    sqn = sqn_f.reshape(NW, K, 16)
    counts = counts_f.reshape(NW, K)
    sidx2 = sort_idx.reshape(N // LC, LC)
    midx2 = match_idx.reshape(N // LC, LC)
    ns, nm, dv = _match_tc(spikes, sidx2, midx2, sums, sqn, counts)
    return (ns.reshape(N), nm.reshape(N), dv.reshape(N))
